# Initial kernel scaffold; baseline (speedup 1.0000x reference)
import functools

import jax
import jax.numpy as jnp
from jax import lax
from jax.experimental import pallas as pl
from jax.experimental.pallas import tpu as pltpu
from jax.experimental.pallas import tpu_sc as plsc

_NC = 2
_NS = 16
_C = 30
_K = 4
_D = 128
_L = 16


def _body(nchunks, x_hbm, idxf_hbm, out_hbm, idx_v, rows_v, out_v, sem):
    wid = lax.axis_index("s") * _NC + lax.axis_index("c")

    @pl.loop(wid, nchunks, step=_NC * _NS)
    def _chunk(c):
        pltpu.sync_copy(idxf_hbm.at[pl.ds(c * (_K * _C), _K * _C)], idx_v)
        pltpu.async_copy(x_hbm.at[idx_v], rows_v, sem).wait()

        @pl.loop(0, _C)
        def _row(i):
            for j in range(_D // _L):
                s = pl.ds(j * _L, _L)
                out_v[i, s] = (rows_v[_K * i, s] * rows_v[_K * i + 1, s]
                               * rows_v[_K * i + 2, s] * rows_v[_K * i + 3, s])

        pltpu.sync_copy(out_v, out_hbm.at[pl.ds(c * _C, _C)])


def kernel(x, indices):
    m, k = indices.shape
    d = x.shape[1]
    assert k == _K and d == _D and m % _C == 0
    idx_flat = indices.reshape(-1)
    mesh = plsc.VectorSubcoreMesh(core_axis_name="c", subcore_axis_name="s")
    f = pl.kernel(
        functools.partial(_body, m // _C),
        out_type=jax.ShapeDtypeStruct((m, d), jnp.float32),
        mesh=mesh,
        scratch_types=[
            pltpu.VMEM((_K * _C,), jnp.int32),
            pltpu.VMEM((_K * _C, _D), jnp.float32),
            pltpu.VMEM((_C, _D), jnp.float32),
            pltpu.SemaphoreType.DMA,
        ],
    )
    return f(x, idx_flat)


# speedup vs baseline: 3.6509x; 3.6509x over previous
import functools

import jax
import jax.numpy as jnp
from jax import lax
from jax.experimental import pallas as pl
from jax.experimental.pallas import tpu as pltpu
from jax.experimental.pallas import tpu_sc as plsc

_NC = 2
_NS = 16
_C = 24
_K = 4
_D = 128
_L = 16


def _body(nchunks, x_hbm, idxf_hbm, out_hbm, idx_v, rows_v, out_v, sem):
    wid = lax.axis_index("s") * _NC + lax.axis_index("c")

    @pl.loop(wid, nchunks, step=_NC * _NS)
    def _chunk(c):
        pltpu.sync_copy(idxf_hbm.at[pl.ds(c * (_K * _C), _K * _C)], idx_v)
        pltpu.async_copy(x_hbm.at[idx_v], rows_v, sem).wait()

        @pl.loop(0, _C)
        def _row(i):
            for j in range(_D // _L):
                s = pl.ds(j * _L, _L)
                out_v[i, s] = (rows_v[_K * i, s] * rows_v[_K * i + 1, s]
                               * rows_v[_K * i + 2, s] * rows_v[_K * i + 3, s])

        pltpu.sync_copy(out_v, out_hbm.at[pl.ds(c * _C, _C)])


def kernel(x, indices):
    m, k = indices.shape
    d = x.shape[1]
    assert k == _K and d == _D and m % _C == 0
    idx_flat = indices.reshape(-1)
    mesh = plsc.VectorSubcoreMesh(core_axis_name="c", subcore_axis_name="s")
    f = pl.kernel(
        functools.partial(_body, m // _C),
        out_type=jax.ShapeDtypeStruct((m, d), jnp.float32),
        mesh=mesh,
        scratch_types=[
            pltpu.VMEM((_K * _C,), jnp.int32),
            pltpu.VMEM((_K * _C, _D), jnp.float32),
            pltpu.VMEM((_C, _D), jnp.float32),
            pltpu.SemaphoreType.DMA,
        ],
    )
    return f(x, idx_flat)


# speedup vs baseline: 6.7708x; 1.8546x over previous
import jax
import jax.numpy as jnp
from jax import lax
from jax.experimental import pallas as pl
from jax.experimental.pallas import tpu as pltpu
from jax.experimental.pallas import tpu_sc as plsc

_NC = 2
_NS = 16
_NB = 4
_C = 24
_K = 4
_D = 128
_L = 16
_G = _K * _C
_NCHUNK = 6250
_QLO = _NCHUNK // (_NC * _NS)
_REM = _NCHUNK - _QLO * (_NC * _NS)
_QHI = _QLO + 1


def _body(x_hbm, idxf_hbm, out_hbm, idx_all,
          r0, r1, r2, r3, o0, o1, o2, o3,
          g0, g1, g2, g3, s0, s1, s2, s3):
    rows = [r0, r1, r2, r3]
    outs = [o0, o1, o2, o3]
    gsem = [g0, g1, g2, g3]
    osem = [s0, s1, s2, s3]
    w = lax.axis_index("s") * _NC + lax.axis_index("c")
    base = w * _QLO + jnp.minimum(w, _REM)
    nw = _QLO + jnp.where(w < _REM, 1, 0)

    pltpu.sync_copy(idxf_hbm.at[pl.ds(base * _G, _QLO * _G)],
                    idx_all.at[pl.ds(0, _QLO * _G)])

    @pl.when(w < _REM)
    def _():
        pltpu.sync_copy(idxf_hbm.at[pl.ds((base + _QLO) * _G, _G)],
                        idx_all.at[pl.ds(_QLO * _G, _G)])

    def gather_start(t, b):
        pltpu.async_copy(x_hbm.at[idx_all.at[pl.ds(t * _G, _G)]],
                         rows[b], gsem[b])

    def gather_wait(b):
        pltpu.make_async_copy(x_hbm.at[pl.ds(0, _G)], rows[b], gsem[b]).wait()

    def out_wait(b):
        pltpu.make_async_copy(outs[b], out_hbm.at[pl.ds(0, _C)], osem[b]).wait()

    for b in range(_NB):
        gather_start(jnp.int32(b), b)

    @pl.loop(0, nw, step=_NB)
    def _group(g):
        for b in range(_NB):
            t = g + b

            @pl.when(t < nw)
            def _():
                gather_wait(b)

                @pl.when(t >= _NB)
                def _():
                    out_wait(b)

                @pl.loop(0, _C)
                def _row(i):
                    for j in range(_D // _L):
                        s = pl.ds(j * _L, _L)
                        outs[b][i, s] = (rows[b][_K * i, s]
                                         * rows[b][_K * i + 1, s]
                                         * rows[b][_K * i + 2, s]
                                         * rows[b][_K * i + 3, s])

                @pl.when(t + _NB < nw)
                def _():
                    gather_start(t + _NB, b)

                pltpu.async_copy(outs[b], out_hbm.at[pl.ds((base + t) * _C, _C)],
                                 osem[b])

    for b in range(_NB):
        out_wait(b)


def kernel(x, indices):
    m, k = indices.shape
    d = x.shape[1]
    assert k == _K and d == _D and m == _NCHUNK * _C
    idx_flat = indices.reshape(-1)
    mesh = plsc.VectorSubcoreMesh(core_axis_name="c", subcore_axis_name="s")
    f = pl.kernel(
        _body,
        out_type=jax.ShapeDtypeStruct((m, d), jnp.float32),
        mesh=mesh,
        scratch_types=(
            [pltpu.VMEM((_QHI * _G,), jnp.int32)]
            + [pltpu.VMEM((_G, _D), jnp.float32) for _ in range(_NB)]
            + [pltpu.VMEM((_C, _D), jnp.float32) for _ in range(_NB)]
            + [pltpu.SemaphoreType.DMA for _ in range(2 * _NB)]
        ),
    )
    return f(x, idx_flat)
